# baseline (speedup 1.0000x reference)
import functools

import jax
import jax.numpy as jnp
from jax import lax
from jax.experimental import pallas as pl
from jax.experimental.pallas import tpu as pltpu
from jax.experimental.pallas import tpu_sc as plsc

_CH = 2000
_RING = 5


def _phase(pos_hbm, idx_hbm, out_hbm, spm, table_v, ibufs, isems,
           obufs, osems, psema, psemb, base, half_off, num_pins, ncyc):
    half = num_pins // 2

    for t in range(_RING):
        pltpu.async_copy(
            idx_hbm.at[pl.ds(base + t * _CH, _CH)], ibufs[t], isems[t])

    pull_a = pltpu.async_copy(spm, table_v.at[pl.ds(0, half)], psema)
    pull_b = pltpu.async_copy(
        pos_hbm.at[pl.ds(half_off + half, half)],
        table_v.at[pl.ds(half, half)], psemb)
    pull_a.wait()
    pull_b.wait()

    @pl.loop(0, ncyc)
    def _(j):
        jbase = base + j * (_RING * _CH)
        for t in range(_RING):
            off = jbase + t * _CH
            pltpu.make_async_copy(
                idx_hbm.at[pl.ds(off, _CH)], ibufs[t], isems[t]).wait()

            @pl.when(j > 0)
            def _():
                pltpu.make_async_copy(
                    obufs[t], out_hbm.at[pl.ds(off, _CH)], osems[t]).wait()

            idx_v, out_v = ibufs[t], obufs[t]

            @plsc.parallel_loop(0, _CH // 16, 1, unroll=8)
            def _(i):
                idx = idx_v[pl.ds(i * 16, 16)]
                out_v[pl.ds(i * 16, 16)] = plsc.load_gather(table_v, [idx])

            pltpu.async_copy(out_v, out_hbm.at[pl.ds(off, _CH)], osems[t])

            @pl.when(j + 1 < ncyc)
            def _():
                pltpu.async_copy(
                    idx_hbm.at[pl.ds(off + _RING * _CH, _CH)],
                    ibufs[t], isems[t])

    for t in range(_RING):
        pltpu.make_async_copy(
            obufs[t], out_hbm.at[pl.ds(0, _CH)], osems[t]).wait()


def _gather_body(pos_hbm, px_hbm, py_hbm, outx_hbm, outy_hbm,
                 table_v, spm,
                 i0, i1, i2, i3, i4, o0, o1, o2, o3, o4,
                 tsem, pa, pb, is0, is1, is2, is3, is4,
                 os0, os1, os2, os3, os4,
                 *, num_pins, per_tile):
    c = lax.axis_index("c")
    s = lax.axis_index("s")
    base = s * per_tile
    ncyc = per_tile // (_RING * _CH)
    ibufs = [i0, i1, i2, i3, i4]
    isems = [is0, is1, is2, is3, is4]
    obufs = [o0, o1, o2, o3, o4]
    osems = [os0, os1, os2, os3, os4]
    half = num_pins // 2

    @pl.when(c == 0)
    def _():
        st = pltpu.make_async_copy(pos_hbm.at[pl.ds(0, half)], spm, tsem)

        @pl.when(s == 0)
        def _():
            st.start()
            st.wait()

        plsc.subcore_barrier()
        _phase(pos_hbm, px_hbm, outx_hbm, spm, table_v, ibufs, isems,
               obufs, osems, pa, pb, base, 0, num_pins, ncyc)

    @pl.when(c == 1)
    def _():
        st = pltpu.make_async_copy(
            pos_hbm.at[pl.ds(num_pins, half)], spm, tsem)

        @pl.when(s == 0)
        def _():
            st.start()
            st.wait()

        plsc.subcore_barrier()
        _phase(pos_hbm, py_hbm, outy_hbm, spm, table_v, ibufs, isems,
               obufs, osems, pa, pb, base, num_pins, num_pins, ncyc)


def kernel(pos, pin_relate_x, pin_relate_y, num_vertices):
    num_pins = pos.shape[0] // 2
    nv = pin_relate_x.shape[0]
    per_tile = nv // 16
    mesh = plsc.VectorSubcoreMesh(core_axis_name="c", subcore_axis_name="s")
    f = pl.kernel(
        functools.partial(_gather_body, num_pins=num_pins, per_tile=per_tile),
        out_type=(jax.ShapeDtypeStruct((nv,), jnp.float32),
                  jax.ShapeDtypeStruct((nv,), jnp.float32)),
        mesh=mesh,
        scratch_types=(
            [pltpu.VMEM((num_pins,), jnp.float32),
             pltpu.VMEM_SHARED((num_pins // 2,), jnp.float32)]
            + [pltpu.VMEM((_CH,), jnp.int32) for _ in range(_RING)]
            + [pltpu.VMEM((_CH,), jnp.float32) for _ in range(_RING)]
            + [pltpu.SemaphoreType.DMA for _ in range(3 + 2 * _RING)]
        ),
        compiler_params=pltpu.CompilerParams(
            needs_layout_passes=False, use_tc_tiling_on_sc=False,
            disable_bounds_checks=True, disable_semaphore_checks=True,
            skip_device_barrier=True),
    )
    return f(pos, pin_relate_x, pin_relate_y)
